# Initial kernel scaffold; baseline (speedup 1.0000x reference)
import jax
import jax.numpy as jnp
from jax.experimental import pallas as pl


def kernel(x, pos, params, edge_index, batch):
    raise NotImplementedError("write your pallas kernel here")


# speedup vs baseline: 3.5072x; 3.5072x over previous
import jax
import jax.numpy as jnp
from jax import lax
from jax.experimental import pallas as pl
from jax.experimental.pallas import tpu as pltpu
from jax.experimental.pallas import tpu_sc as plsc

N = 10000
E = 320000
EMB = 128
NC = 2
NS = 16
NW = NC * NS
K = 128
CJ = 79
ET = CJ * K
EPAD = NW * ET
NA = 10240
NT = NA // NS
NZ = 128
NG = 64


def _ln(x, g, b):
    m = jnp.mean(x, axis=-1, keepdims=True)
    v = jnp.var(x, axis=-1, keepdims=True)
    return (x - m) / jnp.sqrt(v + 1e-5) * g + b


def _sc_gather_body(ta, tb, dst3, src3, pt, gd, gs, pdt,
                    idxd, idxs, bufa, bufb, pdbuf, pxv, pyv, pzv):
    c = lax.axis_index("c")
    s = lax.axis_index("s")
    w = c * NS + s
    pltpu.sync_copy(dst3.at[w], idxd)
    pltpu.sync_copy(src3.at[w], idxs)
    pltpu.sync_copy(pt.at[0], pxv)
    pltpu.sync_copy(pt.at[1], pyv)
    pltpu.sync_copy(pt.at[2], pzv)

    def body(j, carry):
        ebase = w * ET + j * K
        pltpu.sync_copy(ta.at[idxd.at[j]], bufa)
        pltpu.sync_copy(tb.at[idxs.at[j]], bufb)
        for v in range(K // 16):
            di = idxd[j, pl.ds(v * 16, 16)]
            si = idxs[j, pl.ds(v * 16, 16)]
            dx = plsc.load_gather(pxv, [di]) - plsc.load_gather(pxv, [si])
            dy = plsc.load_gather(pyv, [di]) - plsc.load_gather(pyv, [si])
            dz = plsc.load_gather(pzv, [di]) - plsc.load_gather(pzv, [si])
            pdbuf[0, pl.ds(v * 16, 16)] = dx
            pdbuf[1, pl.ds(v * 16, 16)] = dy
            pdbuf[2, pl.ds(v * 16, 16)] = dz
            pdbuf[3, pl.ds(v * 16, 16)] = dx * dx + dy * dy + dz * dz
        pltpu.sync_copy(bufa, gd.at[pl.ds(ebase, K)])
        pltpu.sync_copy(bufb, gs.at[pl.ds(ebase, K)])
        pltpu.sync_copy(pdbuf, pdt.at[:, pl.ds(ebase, K)])
        return carry

    lax.fori_loop(0, CJ, body, 0)


def _sc_gather(ta, tb, dst3, src3, pt):
    f = pl.kernel(
        _sc_gather_body,
        out_type=(
            jax.ShapeDtypeStruct((EPAD, EMB), jnp.float32),
            jax.ShapeDtypeStruct((EPAD, EMB), jnp.float32),
            jax.ShapeDtypeStruct((4, EPAD), jnp.float32),
        ),
        mesh=plsc.VectorSubcoreMesh(core_axis_name="c", subcore_axis_name="s"),
        compiler_params=pltpu.CompilerParams(needs_layout_passes=False),
        scratch_types=[
            pltpu.VMEM((CJ, K), jnp.int32),
            pltpu.VMEM((CJ, K), jnp.int32),
            pltpu.VMEM((K, EMB), jnp.float32),
            pltpu.VMEM((K, EMB), jnp.float32),
            pltpu.VMEM((4, K), jnp.float32),
            pltpu.VMEM((NA,), jnp.float32),
            pltpu.VMEM((NA,), jnp.float32),
            pltpu.VMEM((NA,), jnp.float32),
        ],
    )
    return f(ta, tb, dst3, src3, pt)


def _sc_scatter_msg_body(msg, dst3, out, idxd, rbuf, acc):
    c = lax.axis_index("c")
    s = lax.axis_index("s")
    w = c * NS + s

    def zb(k, carry):
        rbuf[k // (EMB // 16), pl.ds((k % (EMB // 16)) * 16, 16)] = jnp.zeros(
            (16,), jnp.float32)
        return carry

    lax.fori_loop(0, NZ * (EMB // 16), zb, 0)

    def zc(q, carry):
        pltpu.sync_copy(rbuf, acc.at[pl.ds(s * NT + q * NZ, NZ)])
        return carry

    lax.fori_loop(0, NT // NZ, zc, 0)
    plsc.subcore_barrier()

    pltpu.sync_copy(dst3.at[w], idxd)

    def body(j, carry):
        ebase = w * ET + j * K
        pltpu.sync_copy(msg.at[pl.ds(ebase, K)], rbuf)
        pltpu.sync_copy(rbuf, acc.at[idxd.at[j]], add=True)
        return carry

    lax.fori_loop(0, CJ, body, 0)
    plsc.subcore_barrier()

    def dump(q, carry):
        off = s * NT + q * NZ
        pltpu.sync_copy(acc.at[pl.ds(off, NZ)], rbuf)
        pltpu.sync_copy(rbuf, out.at[c, pl.ds(off, NZ)])
        return carry

    lax.fori_loop(0, NT // NZ, dump, 0)


def _sc_scatter_msg(msg, dst3):
    f = pl.kernel(
        _sc_scatter_msg_body,
        out_type=jax.ShapeDtypeStruct((NC, NA, EMB), jnp.float32),
        mesh=plsc.VectorSubcoreMesh(core_axis_name="c", subcore_axis_name="s"),
        compiler_params=pltpu.CompilerParams(needs_layout_passes=False),
        scratch_types=[
            pltpu.VMEM((CJ, K), jnp.int32),
            pltpu.VMEM((K, EMB), jnp.float32),
            pltpu.VMEM_SHARED((NA, EMB), jnp.float32),
        ],
    )
    return f(msg, dst3)


def _sc_scatter_pos_body(pmt, dst3, pout, idxd, pmbuf, acc4):
    c = lax.axis_index("c")
    s = lax.axis_index("s")
    w = c * NS + s

    def za(k, carry):
        acc4[k // (NA // 16), pl.ds((k % (NA // 16)) * 16, 16)] = jnp.zeros(
            (16,), jnp.float32)
        return carry

    lax.fori_loop(0, 4 * (NA // 16), za, 0)

    pltpu.sync_copy(dst3.at[w], idxd)
    rows = [jnp.full((16,), r, jnp.int32) for r in range(4)]

    def body(j, carry):
        ebase = w * ET + j * K
        pltpu.sync_copy(pmt.at[:, pl.ds(ebase, K)], pmbuf)
        for v in range(K // 16):
            di = idxd[j, pl.ds(v * 16, 16)]
            for r in range(4):
                plsc.addupdate_scatter(acc4, [rows[r], di],
                                       pmbuf[r, pl.ds(v * 16, 16)])
        return carry

    lax.fori_loop(0, CJ, body, 0)
    pltpu.sync_copy(acc4, pout.at[w])


def _sc_scatter_pos(pmt, dst3):
    f = pl.kernel(
        _sc_scatter_pos_body,
        out_type=jax.ShapeDtypeStruct((NW, 4, NA), jnp.float32),
        mesh=plsc.VectorSubcoreMesh(core_axis_name="c", subcore_axis_name="s"),
        compiler_params=pltpu.CompilerParams(needs_layout_passes=False),
        scratch_types=[
            pltpu.VMEM((CJ, K), jnp.int32),
            pltpu.VMEM((4, K), jnp.float32),
            pltpu.VMEM((4, NA), jnp.float32),
        ],
    )
    return f(pmt, dst3)


_NB = 1024
_EB = 2048


def _embed_pre_body(x, embW, embb, wd, ws, h, ta, tb):
    hv = jnp.dot(x[...], embW[...], preferred_element_type=jnp.float32) + embb[...]
    h[...] = hv
    ta[...] = jnp.dot(hv, wd[...], preferred_element_type=jnp.float32)
    tb[...] = jnp.dot(hv, ws[...], preferred_element_type=jnp.float32)


def _embed_pre(x, embW, embb, wd, ws):
    full = lambda i: (0, 0)
    return pl.pallas_call(
        _embed_pre_body,
        grid=(NA // _NB,),
        in_specs=[
            pl.BlockSpec((_NB, EMB), lambda i: (i, 0)),
            pl.BlockSpec((EMB, EMB), full),
            pl.BlockSpec((1, EMB), full),
            pl.BlockSpec((EMB, EMB), full),
            pl.BlockSpec((EMB, EMB), full),
        ],
        out_specs=[
            pl.BlockSpec((_NB, EMB), lambda i: (i, 0)),
            pl.BlockSpec((_NB, EMB), lambda i: (i, 0)),
            pl.BlockSpec((_NB, EMB), lambda i: (i, 0)),
        ],
        out_shape=[
            jax.ShapeDtypeStruct((NA, EMB), jnp.float32),
            jax.ShapeDtypeStruct((NA, EMB), jnp.float32),
            jax.ShapeDtypeStruct((NA, EMB), jnp.float32),
        ],
    )(x, embW, embb, wd, ws)


def _edge_body(gdr, gsr, pdtr, w1c, b1, g1, be1, w2, b2, g2, be2,
               pw1, pb1, pg1, pbe1, pw2, pb2, msg_o, pmt_o):
    g = gdr[...] + gsr[...]
    pdt = pdtr[...]
    d = jnp.sqrt(pdt[3:4, :] + 1e-12)
    x1 = g + lax.dot_general(d, w1c[...], (((0,), (0,)), ((), ())),
                             preferred_element_type=jnp.float32) + b1[...]
    x1 = jax.nn.relu(_ln(x1, g1[...], be1[...]))
    x2 = jnp.dot(x1, w2[...], preferred_element_type=jnp.float32) + b2[...]
    msg = jax.nn.relu(_ln(x2, g2[...], be2[...]))
    p1 = jnp.dot(msg, pw1[...], preferred_element_type=jnp.float32) + pb1[...]
    p1 = jax.nn.relu(_ln(p1, pg1[...], pbe1[...]))
    pw = jnp.dot(p1, pw2[...], preferred_element_type=jnp.float32) + pb2[...]
    pwt = jnp.transpose(pw)
    row = lax.broadcasted_iota(jnp.int32, (4, pdt.shape[1]), 0)
    pm = jnp.where(row < 3, pdt * pwt, 1.0)
    msg_o[...] = msg
    pmt_o[...] = pm


def _edge(gd, gs, pdt, p):
    full = lambda i: (0, 0)
    return pl.pallas_call(
        _edge_body,
        grid=(EPAD // _EB,),
        in_specs=[
            pl.BlockSpec((_EB, EMB), lambda i: (i, 0)),
            pl.BlockSpec((_EB, EMB), lambda i: (i, 0)),
            pl.BlockSpec((4, _EB), lambda i: (0, i)),
        ] + [pl.BlockSpec(w.shape, full) for w in p],
        out_specs=[
            pl.BlockSpec((_EB, EMB), lambda i: (i, 0)),
            pl.BlockSpec((4, _EB), lambda i: (0, i)),
        ],
        out_shape=[
            jax.ShapeDtypeStruct((EPAD, EMB), jnp.float32),
            jax.ShapeDtypeStruct((4, EPAD), jnp.float32),
        ],
    )(gd, gs, pdt, *p)


def _node_common(h, a0r, a1r, uw1h, uw1m, ub1, ug1, ube1, uw2, ub2, ug2, ube2):
    ma = a0r[0] + a1r[0]
    u = jnp.dot(h, uw1h[...], preferred_element_type=jnp.float32) \
        + jnp.dot(ma, uw1m[...], preferred_element_type=jnp.float32) + ub1[...]
    u = jax.nn.relu(_ln(u, ug1[...], ube1[...]))
    u = jnp.dot(u, uw2[...], preferred_element_type=jnp.float32) + ub2[...]
    u = jax.nn.relu(_ln(u, ug2[...], ube2[...]))
    return h + u


def _pos_update(ptr, p0r):
    pa = jnp.sum(p0r[...], axis=0)
    cnt = jnp.maximum(pa[3:4, :], 1.0)
    row = lax.broadcasted_iota(jnp.int32, (8, pa.shape[1]), 0)
    delta = jnp.where(row < 3, jnp.concatenate([pa, pa], axis=0) / cnt, 0.0)
    return ptr[...] + delta


def _node_body(hr, ptr, a0r, a1r, p0r, uw1h, uw1m, ub1, ug1, ube1,
               uw2, ub2, ug2, ube2, wd, ws, hn, ptn, ta, tb):
    hv = _node_common(hr[...], a0r, a1r, uw1h, uw1m, ub1, ug1, ube1,
                      uw2, ub2, ug2, ube2)
    hn[...] = hv
    ptn[...] = _pos_update(ptr, p0r)
    ta[...] = jnp.dot(hv, wd[...], preferred_element_type=jnp.float32)
    tb[...] = jnp.dot(hv, ws[...], preferred_element_type=jnp.float32)


def _node(h, pt, agg, pagg, p, wd, ws):
    full = lambda i: (0, 0)
    return pl.pallas_call(
        _node_body,
        grid=(NA // _NB,),
        in_specs=[
            pl.BlockSpec((_NB, EMB), lambda i: (i, 0)),
            pl.BlockSpec((8, _NB), lambda i: (0, i)),
            pl.BlockSpec((1, _NB, EMB), lambda i: (0, i, 0)),
            pl.BlockSpec((1, _NB, EMB), lambda i: (0, i, 0)),
            pl.BlockSpec((NW, 4, _NB), lambda i: (0, 0, i)),
        ] + [pl.BlockSpec(w.shape, full) for w in p]
          + [pl.BlockSpec((EMB, EMB), full)] * 2,
        out_specs=[
            pl.BlockSpec((_NB, EMB), lambda i: (i, 0)),
            pl.BlockSpec((8, _NB), lambda i: (0, i)),
            pl.BlockSpec((_NB, EMB), lambda i: (i, 0)),
            pl.BlockSpec((_NB, EMB), lambda i: (i, 0)),
        ],
        out_shape=[
            jax.ShapeDtypeStruct((NA, EMB), jnp.float32),
            jax.ShapeDtypeStruct((8, NA), jnp.float32),
            jax.ShapeDtypeStruct((NA, EMB), jnp.float32),
            jax.ShapeDtypeStruct((NA, EMB), jnp.float32),
        ],
    )(h, pt, agg[0:1], agg[1:2], pagg, *p, wd, ws)


def _node_last_body(hr, ptr, a0r, a1r, p0r, batchr, uw1h, uw1m,
                    ub1, ug1, ube1, uw2, ub2, ug2, ube2, hn, ptn, gemb):
    i = pl.program_id(0)
    hv = _node_common(hr[...], a0r, a1r, uw1h, uw1m, ub1, ug1, ube1,
                      uw2, ub2, ug2, ube2)
    hn[...] = hv
    ptn[...] = _pos_update(ptr, p0r)
    bb = batchr[0, 0, :]
    row = lax.broadcasted_iota(jnp.int32, (NG, _NB), 0)
    onehot = jnp.where(row == bb[None, :], 1.0, 0.0)
    contrib = jnp.dot(onehot, hv, preferred_element_type=jnp.float32)

    @pl.when(i == 0)
    def _():
        gemb[...] = jnp.zeros_like(gemb)

    gemb[...] += contrib


def _node_last(h, pt, agg, pagg, p, batch3d):
    full = lambda i: (0, 0)
    return pl.pallas_call(
        _node_last_body,
        grid=(NA // _NB,),
        in_specs=[
            pl.BlockSpec((_NB, EMB), lambda i: (i, 0)),
            pl.BlockSpec((8, _NB), lambda i: (0, i)),
            pl.BlockSpec((1, _NB, EMB), lambda i: (0, i, 0)),
            pl.BlockSpec((1, _NB, EMB), lambda i: (0, i, 0)),
            pl.BlockSpec((NW, 4, _NB), lambda i: (0, 0, i)),
            pl.BlockSpec((1, 1, _NB), lambda i: (i, 0, 0)),
        ] + [pl.BlockSpec(w.shape, full) for w in p],
        out_specs=[
            pl.BlockSpec((_NB, EMB), lambda i: (i, 0)),
            pl.BlockSpec((8, _NB), lambda i: (0, i)),
            pl.BlockSpec((NG, EMB), full),
        ],
        out_shape=[
            jax.ShapeDtypeStruct((NA, EMB), jnp.float32),
            jax.ShapeDtypeStruct((8, NA), jnp.float32),
            jax.ShapeDtypeStruct((NG, EMB), jnp.float32),
        ],
    )(h, pt, agg[0:1], agg[1:2], pagg, batch3d, *p)


def _v(a):
    return a.reshape(1, -1)


def kernel(x, pos, params, edge_index, batch):
    src_p = jnp.full((EPAD,), 0, jnp.int32).at[:E].set(edge_index[0])
    dst_p = jnp.full((EPAD,), NA - 1, jnp.int32).at[:E].set(edge_index[1])
    src3 = src_p.reshape(NW, CJ, K)
    dst3 = dst_p.reshape(NW, CJ, K)
    xp = jnp.pad(x, ((0, NA - N), (0, 0)))
    pt = jnp.pad(pos.T, ((0, 5), (0, NA - N)))
    batch3d = jnp.pad(batch, (0, NA - N), constant_values=NG).reshape(
        NA // _NB, 1, _NB)

    layers = params['layers']

    def msg_split(p):
        return p['msg_W1'][:EMB], p['msg_W1'][EMB:2 * EMB]

    wd0, ws0 = msg_split(layers[0])
    h, ta, tb = _embed_pre(xp, params['emb_W'], _v(params['emb_b']), wd0, ws0)

    for l, p in enumerate(layers):
        edge_w = [
            p['msg_W1'][2 * EMB:2 * EMB + 1], _v(p['msg_b1']),
            _v(p['msg_g1']), _v(p['msg_be1']),
            p['msg_W2'], _v(p['msg_b2']), _v(p['msg_g2']), _v(p['msg_be2']),
            p['pos_W1'], _v(p['pos_b1']), _v(p['pos_g1']), _v(p['pos_be1']),
            p['pos_W2'], _v(p['pos_b2']),
        ]
        node_w = [
            p['upd_W1'][:EMB], p['upd_W1'][EMB:],
            _v(p['upd_b1']), _v(p['upd_g1']), _v(p['upd_be1']),
            p['upd_W2'], _v(p['upd_b2']), _v(p['upd_g2']), _v(p['upd_be2']),
        ]
        gd, gs, pdt = _sc_gather(ta, tb, dst3, src3, pt)
        msg, pmt = _edge(gd, gs, pdt, edge_w)
        agg = _sc_scatter_msg(msg, dst3)
        pagg = _sc_scatter_pos(pmt, dst3)
        if l + 1 < len(layers):
            wd, ws = msg_split(layers[l + 1])
            h, pt, ta, tb = _node(h, pt, agg, pagg, node_w, wd, ws)
        else:
            h, pt, gemb = _node_last(h, pt, agg, pagg, node_w, batch3d)

    return h[:N], gemb, pt[:3, :N].T


# speedup vs baseline: 3.6659x; 1.0452x over previous
import jax
import jax.numpy as jnp
from jax import lax
from jax.experimental import pallas as pl
from jax.experimental.pallas import tpu as pltpu
from jax.experimental.pallas import tpu_sc as plsc

N = 10000
E = 320000
EMB = 128
NC = 2
NS = 16
NW = NC * NS
K = 128
CJ = 80
ET = CJ * K
EPAD = NW * ET
NA = 10240
NT = NA // NS
NZ = 128
NG = 64


def _ln(x, g, b):
    m = jnp.mean(x, axis=-1, keepdims=True)
    v = jnp.var(x, axis=-1, keepdims=True)
    return (x - m) / jnp.sqrt(v + 1e-5) * g + b


def _sc_gather_body(ta, tb, dst3, src3, pt, gd, gs, pdt,
                    idxd, idxs, bufa0, bufb0, pd0, bufa1, bufb1, pd1,
                    pxv, pyv, pzv, ga0, ga1, wr0, wr1):
    c = lax.axis_index("c")
    s = lax.axis_index("s")
    w = c * NS + s
    pltpu.sync_copy(dst3.at[w], idxd)
    pltpu.sync_copy(src3.at[w], idxs)
    pltpu.sync_copy(pt.at[0], pxv)
    pltpu.sync_copy(pt.at[1], pyv)
    pltpu.sync_copy(pt.at[2], pzv)

    sets = ((bufa0, bufb0, pd0, ga0, wr0), (bufa1, bufb1, pd1, ga1, wr1))

    def fire_gathers(j, st):
        bufa, bufb, _, ga, _ = st
        pltpu.async_copy(ta.at[idxd.at[j]], bufa, ga)
        pltpu.async_copy(tb.at[idxs.at[j]], bufb, ga)

    def wait_gathers(st):
        bufa, bufb, _, ga, _ = st
        pltpu.make_async_copy(ta.at[idxd.at[0]], bufa, ga).wait()
        pltpu.make_async_copy(tb.at[idxs.at[0]], bufb, ga).wait()

    def fire_writes(j, st):
        bufa, bufb, pdb, _, wr = st
        ebase = w * ET + j * K
        pltpu.async_copy(bufa, gd.at[pl.ds(ebase, K)], wr)
        pltpu.async_copy(bufb, gs.at[pl.ds(ebase, K)], wr)
        pltpu.async_copy(pdb, pdt.at[:, pl.ds(ebase, K)], wr)

    def wait_writes(st):
        bufa, bufb, pdb, _, wr = st
        pltpu.make_async_copy(bufa, gd.at[pl.ds(0, K)], wr).wait()
        pltpu.make_async_copy(bufb, gs.at[pl.ds(0, K)], wr).wait()
        pltpu.make_async_copy(pdb, pdt.at[:, pl.ds(0, K)], wr).wait()

    def compute_pd(j, st):
        pdb = st[2]
        for v in range(K // 16):
            di = idxd[j, pl.ds(v * 16, 16)]
            si = idxs[j, pl.ds(v * 16, 16)]
            dx = plsc.load_gather(pxv, [di]) - plsc.load_gather(pxv, [si])
            dy = plsc.load_gather(pyv, [di]) - plsc.load_gather(pyv, [si])
            dz = plsc.load_gather(pzv, [di]) - plsc.load_gather(pzv, [si])
            pdb[0, pl.ds(v * 16, 16)] = dx
            pdb[1, pl.ds(v * 16, 16)] = dy
            pdb[2, pl.ds(v * 16, 16)] = dz
            pdb[3, pl.ds(v * 16, 16)] = dx * dx + dy * dy + dz * dz

    def body(i, carry):
        j0 = 2 * i
        j1 = 2 * i + 1

        @pl.when(i > 0)
        def _():
            wait_writes(sets[0])

        fire_gathers(j0, sets[0])
        compute_pd(j0, sets[0])

        @pl.when(i > 0)
        def _():
            wait_writes(sets[1])

        fire_gathers(j1, sets[1])
        compute_pd(j1, sets[1])
        wait_gathers(sets[0])
        fire_writes(j0, sets[0])
        wait_gathers(sets[1])
        fire_writes(j1, sets[1])
        return carry

    lax.fori_loop(0, CJ // 2, body, 0)
    wait_writes(sets[0])
    wait_writes(sets[1])


def _sc_gather(ta, tb, dst3, src3, pt):
    f = pl.kernel(
        _sc_gather_body,
        out_type=(
            jax.ShapeDtypeStruct((EPAD, EMB), jnp.float32),
            jax.ShapeDtypeStruct((EPAD, EMB), jnp.float32),
            jax.ShapeDtypeStruct((4, EPAD), jnp.float32),
        ),
        mesh=plsc.VectorSubcoreMesh(core_axis_name="c", subcore_axis_name="s"),
        compiler_params=pltpu.CompilerParams(needs_layout_passes=False),
        scratch_types=[
            pltpu.VMEM((CJ, K), jnp.int32),
            pltpu.VMEM((CJ, K), jnp.int32),
            pltpu.VMEM((K, EMB), jnp.float32),
            pltpu.VMEM((K, EMB), jnp.float32),
            pltpu.VMEM((4, K), jnp.float32),
            pltpu.VMEM((K, EMB), jnp.float32),
            pltpu.VMEM((K, EMB), jnp.float32),
            pltpu.VMEM((4, K), jnp.float32),
            pltpu.VMEM((NA,), jnp.float32),
            pltpu.VMEM((NA,), jnp.float32),
            pltpu.VMEM((NA,), jnp.float32),
            pltpu.SemaphoreType.DMA,
            pltpu.SemaphoreType.DMA,
            pltpu.SemaphoreType.DMA,
            pltpu.SemaphoreType.DMA,
        ],
    )
    return f(ta, tb, dst3, src3, pt)


def _sc_scatter_msg_body(msg, dst3, out, idxd, rb0, rb1, ld0, ld1, sc0, sc1,
                         acc):
    c = lax.axis_index("c")
    s = lax.axis_index("s")
    w = c * NS + s

    def zb(k, carry):
        rb0[k // (EMB // 16), pl.ds((k % (EMB // 16)) * 16, 16)] = jnp.zeros(
            (16,), jnp.float32)
        return carry

    lax.fori_loop(0, NZ * (EMB // 16), zb, 0)

    def zc(q, carry):
        pltpu.sync_copy(rb0, acc.at[pl.ds(s * NT + q * NZ, NZ)])
        return carry

    lax.fori_loop(0, NT // NZ, zc, 0)
    plsc.subcore_barrier()

    pltpu.sync_copy(dst3.at[w], idxd)
    sets = ((rb0, ld0, sc0), (rb1, ld1, sc1))

    def fire_load(j, st):
        rb, ld, _ = st
        pltpu.async_copy(msg.at[pl.ds(w * ET + j * K, K)], rb, ld)

    def wait_load(st):
        rb, ld, _ = st
        pltpu.make_async_copy(msg.at[pl.ds(0, K)], rb, ld).wait()

    def fire_add(j, st):
        rb, _, sc = st
        pltpu.async_copy(rb, acc.at[idxd.at[j]], sc, add=True)

    def wait_add(st):
        rb, _, sc = st
        pltpu.make_async_copy(rb, acc.at[idxd.at[0]], sc).wait()

    def body(i, carry):
        j0 = 2 * i
        j1 = 2 * i + 1

        @pl.when(i > 0)
        def _():
            wait_add(sets[0])

        fire_load(j0, sets[0])

        @pl.when(i > 0)
        def _():
            wait_add(sets[1])

        fire_load(j1, sets[1])
        wait_load(sets[0])
        fire_add(j0, sets[0])
        wait_load(sets[1])
        fire_add(j1, sets[1])
        return carry

    lax.fori_loop(0, CJ // 2, body, 0)
    wait_add(sets[0])
    wait_add(sets[1])
    plsc.subcore_barrier()

    def dump(q, carry):
        off = s * NT + q * NZ
        pltpu.sync_copy(acc.at[pl.ds(off, NZ)], rb0)
        pltpu.sync_copy(rb0, out.at[c, pl.ds(off, NZ)])
        return carry

    lax.fori_loop(0, NT // NZ, dump, 0)


def _sc_scatter_msg(msg, dst3):
    f = pl.kernel(
        _sc_scatter_msg_body,
        out_type=jax.ShapeDtypeStruct((NC, NA, EMB), jnp.float32),
        mesh=plsc.VectorSubcoreMesh(core_axis_name="c", subcore_axis_name="s"),
        compiler_params=pltpu.CompilerParams(needs_layout_passes=False),
        scratch_types=[
            pltpu.VMEM((CJ, K), jnp.int32),
            pltpu.VMEM((K, EMB), jnp.float32),
            pltpu.VMEM((K, EMB), jnp.float32),
            pltpu.SemaphoreType.DMA,
            pltpu.SemaphoreType.DMA,
            pltpu.SemaphoreType.DMA,
            pltpu.SemaphoreType.DMA,
            pltpu.VMEM_SHARED((NA, EMB), jnp.float32),
        ],
    )
    return f(msg, dst3)


def _sc_scatter_pos_body(pmt, dst3, pout, idxd, pm0, pm1, ld0, ld1, acc4):
    c = lax.axis_index("c")
    s = lax.axis_index("s")
    w = c * NS + s

    def za(k, carry):
        acc4[k // (NA // 16), pl.ds((k % (NA // 16)) * 16, 16)] = jnp.zeros(
            (16,), jnp.float32)
        return carry

    lax.fori_loop(0, 4 * (NA // 16), za, 0)

    pltpu.sync_copy(dst3.at[w], idxd)
    rows = [jnp.full((16,), r, jnp.int32) for r in range(4)]
    sets = ((pm0, ld0), (pm1, ld1))

    def fire_load(j, st):
        pm, ld = st
        pltpu.async_copy(pmt.at[:, pl.ds(w * ET + j * K, K)], pm, ld)

    def wait_load(st):
        pm, ld = st
        pltpu.make_async_copy(pmt.at[:, pl.ds(0, K)], pm, ld).wait()

    def accum(j, st):
        pm = st[0]
        for v in range(K // 16):
            di = idxd[j, pl.ds(v * 16, 16)]
            for r in range(4):
                plsc.addupdate_scatter(acc4, [rows[r], di],
                                       pm[r, pl.ds(v * 16, 16)])

    def body(i, carry):
        j0 = 2 * i
        j1 = 2 * i + 1
        fire_load(j0, sets[0])
        fire_load(j1, sets[1])
        wait_load(sets[0])
        accum(j0, sets[0])
        wait_load(sets[1])
        accum(j1, sets[1])
        return carry

    lax.fori_loop(0, CJ // 2, body, 0)
    pltpu.sync_copy(acc4, pout.at[w])


def _sc_scatter_pos(pmt, dst3):
    f = pl.kernel(
        _sc_scatter_pos_body,
        out_type=jax.ShapeDtypeStruct((NW, 4, NA), jnp.float32),
        mesh=plsc.VectorSubcoreMesh(core_axis_name="c", subcore_axis_name="s"),
        compiler_params=pltpu.CompilerParams(needs_layout_passes=False),
        scratch_types=[
            pltpu.VMEM((CJ, K), jnp.int32),
            pltpu.VMEM((4, K), jnp.float32),
            pltpu.VMEM((4, K), jnp.float32),
            pltpu.SemaphoreType.DMA,
            pltpu.SemaphoreType.DMA,
            pltpu.VMEM((4, NA), jnp.float32),
        ],
    )
    return f(pmt, dst3)


_NB = 1024
_EB = 2048


def _embed_pre_body(x, embW, embb, wd, ws, h, ta, tb):
    hv = jnp.dot(x[...], embW[...], preferred_element_type=jnp.float32) + embb[...]
    h[...] = hv
    ta[...] = jnp.dot(hv, wd[...], preferred_element_type=jnp.float32)
    tb[...] = jnp.dot(hv, ws[...], preferred_element_type=jnp.float32)


def _embed_pre(x, embW, embb, wd, ws):
    full = lambda i: (0, 0)
    return pl.pallas_call(
        _embed_pre_body,
        grid=(NA // _NB,),
        in_specs=[
            pl.BlockSpec((_NB, EMB), lambda i: (i, 0)),
            pl.BlockSpec((EMB, EMB), full),
            pl.BlockSpec((1, EMB), full),
            pl.BlockSpec((EMB, EMB), full),
            pl.BlockSpec((EMB, EMB), full),
        ],
        out_specs=[
            pl.BlockSpec((_NB, EMB), lambda i: (i, 0)),
            pl.BlockSpec((_NB, EMB), lambda i: (i, 0)),
            pl.BlockSpec((_NB, EMB), lambda i: (i, 0)),
        ],
        out_shape=[
            jax.ShapeDtypeStruct((NA, EMB), jnp.float32),
            jax.ShapeDtypeStruct((NA, EMB), jnp.float32),
            jax.ShapeDtypeStruct((NA, EMB), jnp.float32),
        ],
    )(x, embW, embb, wd, ws)


def _edge_body(gdr, gsr, pdtr, w1c, b1, g1, be1, w2, b2, g2, be2,
               pw1, pb1, pg1, pbe1, pw2, pb2, msg_o, pmt_o):
    g = gdr[...] + gsr[...]
    pdt = pdtr[...]
    d = jnp.sqrt(pdt[3:4, :] + 1e-12)
    x1 = g + lax.dot_general(d, w1c[...], (((0,), (0,)), ((), ())),
                             preferred_element_type=jnp.float32) + b1[...]
    x1 = jax.nn.relu(_ln(x1, g1[...], be1[...]))
    x2 = jnp.dot(x1, w2[...], preferred_element_type=jnp.float32) + b2[...]
    msg = jax.nn.relu(_ln(x2, g2[...], be2[...]))
    p1 = jnp.dot(msg, pw1[...], preferred_element_type=jnp.float32) + pb1[...]
    p1 = jax.nn.relu(_ln(p1, pg1[...], pbe1[...]))
    pw = jnp.dot(p1, pw2[...], preferred_element_type=jnp.float32) + pb2[...]
    pwt = jnp.transpose(pw)
    row = lax.broadcasted_iota(jnp.int32, (4, pdt.shape[1]), 0)
    pm = jnp.where(row < 3, pdt * pwt, 1.0)
    msg_o[...] = msg
    pmt_o[...] = pm


def _edge(gd, gs, pdt, p):
    full = lambda i: (0, 0)
    return pl.pallas_call(
        _edge_body,
        grid=(EPAD // _EB,),
        in_specs=[
            pl.BlockSpec((_EB, EMB), lambda i: (i, 0)),
            pl.BlockSpec((_EB, EMB), lambda i: (i, 0)),
            pl.BlockSpec((4, _EB), lambda i: (0, i)),
        ] + [pl.BlockSpec(w.shape, full) for w in p],
        out_specs=[
            pl.BlockSpec((_EB, EMB), lambda i: (i, 0)),
            pl.BlockSpec((4, _EB), lambda i: (0, i)),
        ],
        out_shape=[
            jax.ShapeDtypeStruct((EPAD, EMB), jnp.float32),
            jax.ShapeDtypeStruct((4, EPAD), jnp.float32),
        ],
    )(gd, gs, pdt, *p)


def _node_common(h, a0r, a1r, uw1h, uw1m, ub1, ug1, ube1, uw2, ub2, ug2, ube2):
    ma = a0r[0] + a1r[0]
    u = jnp.dot(h, uw1h[...], preferred_element_type=jnp.float32) \
        + jnp.dot(ma, uw1m[...], preferred_element_type=jnp.float32) + ub1[...]
    u = jax.nn.relu(_ln(u, ug1[...], ube1[...]))
    u = jnp.dot(u, uw2[...], preferred_element_type=jnp.float32) + ub2[...]
    u = jax.nn.relu(_ln(u, ug2[...], ube2[...]))
    return h + u


def _pos_update(ptr, p0r):
    pa = jnp.sum(p0r[...], axis=0)
    cnt = jnp.maximum(pa[3:4, :], 1.0)
    row = lax.broadcasted_iota(jnp.int32, (8, pa.shape[1]), 0)
    delta = jnp.where(row < 3, jnp.concatenate([pa, pa], axis=0) / cnt, 0.0)
    return ptr[...] + delta


def _node_body(hr, ptr, a0r, a1r, p0r, uw1h, uw1m, ub1, ug1, ube1,
               uw2, ub2, ug2, ube2, wd, ws, hn, ptn, ta, tb):
    hv = _node_common(hr[...], a0r, a1r, uw1h, uw1m, ub1, ug1, ube1,
                      uw2, ub2, ug2, ube2)
    hn[...] = hv
    ptn[...] = _pos_update(ptr, p0r)
    ta[...] = jnp.dot(hv, wd[...], preferred_element_type=jnp.float32)
    tb[...] = jnp.dot(hv, ws[...], preferred_element_type=jnp.float32)


def _node(h, pt, agg, pagg, p, wd, ws):
    full = lambda i: (0, 0)
    return pl.pallas_call(
        _node_body,
        grid=(NA // _NB,),
        in_specs=[
            pl.BlockSpec((_NB, EMB), lambda i: (i, 0)),
            pl.BlockSpec((8, _NB), lambda i: (0, i)),
            pl.BlockSpec((1, _NB, EMB), lambda i: (0, i, 0)),
            pl.BlockSpec((1, _NB, EMB), lambda i: (0, i, 0)),
            pl.BlockSpec((NW, 4, _NB), lambda i: (0, 0, i)),
        ] + [pl.BlockSpec(w.shape, full) for w in p]
          + [pl.BlockSpec((EMB, EMB), full)] * 2,
        out_specs=[
            pl.BlockSpec((_NB, EMB), lambda i: (i, 0)),
            pl.BlockSpec((8, _NB), lambda i: (0, i)),
            pl.BlockSpec((_NB, EMB), lambda i: (i, 0)),
            pl.BlockSpec((_NB, EMB), lambda i: (i, 0)),
        ],
        out_shape=[
            jax.ShapeDtypeStruct((NA, EMB), jnp.float32),
            jax.ShapeDtypeStruct((8, NA), jnp.float32),
            jax.ShapeDtypeStruct((NA, EMB), jnp.float32),
            jax.ShapeDtypeStruct((NA, EMB), jnp.float32),
        ],
    )(h, pt, agg[0:1], agg[1:2], pagg, *p, wd, ws)


def _node_last_body(hr, ptr, a0r, a1r, p0r, batchr, uw1h, uw1m,
                    ub1, ug1, ube1, uw2, ub2, ug2, ube2, hn, ptn, gemb):
    i = pl.program_id(0)
    hv = _node_common(hr[...], a0r, a1r, uw1h, uw1m, ub1, ug1, ube1,
                      uw2, ub2, ug2, ube2)
    hn[...] = hv
    ptn[...] = _pos_update(ptr, p0r)
    bb = batchr[0, 0, :]
    row = lax.broadcasted_iota(jnp.int32, (NG, _NB), 0)
    onehot = jnp.where(row == bb[None, :], 1.0, 0.0)
    contrib = jnp.dot(onehot, hv, preferred_element_type=jnp.float32)

    @pl.when(i == 0)
    def _():
        gemb[...] = jnp.zeros_like(gemb)

    gemb[...] += contrib


def _node_last(h, pt, agg, pagg, p, batch3d):
    full = lambda i: (0, 0)
    return pl.pallas_call(
        _node_last_body,
        grid=(NA // _NB,),
        in_specs=[
            pl.BlockSpec((_NB, EMB), lambda i: (i, 0)),
            pl.BlockSpec((8, _NB), lambda i: (0, i)),
            pl.BlockSpec((1, _NB, EMB), lambda i: (0, i, 0)),
            pl.BlockSpec((1, _NB, EMB), lambda i: (0, i, 0)),
            pl.BlockSpec((NW, 4, _NB), lambda i: (0, 0, i)),
            pl.BlockSpec((1, 1, _NB), lambda i: (i, 0, 0)),
        ] + [pl.BlockSpec(w.shape, full) for w in p],
        out_specs=[
            pl.BlockSpec((_NB, EMB), lambda i: (i, 0)),
            pl.BlockSpec((8, _NB), lambda i: (0, i)),
            pl.BlockSpec((NG, EMB), full),
        ],
        out_shape=[
            jax.ShapeDtypeStruct((NA, EMB), jnp.float32),
            jax.ShapeDtypeStruct((8, NA), jnp.float32),
            jax.ShapeDtypeStruct((NG, EMB), jnp.float32),
        ],
    )(h, pt, agg[0:1], agg[1:2], pagg, batch3d, *p)


def _v(a):
    return a.reshape(1, -1)


def kernel(x, pos, params, edge_index, batch):
    src_p = jnp.full((EPAD,), 0, jnp.int32).at[:E].set(edge_index[0])
    dst_p = jnp.full((EPAD,), NA - 1, jnp.int32).at[:E].set(edge_index[1])
    src3 = src_p.reshape(NW, CJ, K)
    dst3 = dst_p.reshape(NW, CJ, K)
    xp = jnp.pad(x, ((0, NA - N), (0, 0)))
    pt = jnp.pad(pos.T, ((0, 5), (0, NA - N)))
    batch3d = jnp.pad(batch, (0, NA - N), constant_values=NG).reshape(
        NA // _NB, 1, _NB)

    layers = params['layers']

    def msg_split(p):
        return p['msg_W1'][:EMB], p['msg_W1'][EMB:2 * EMB]

    wd0, ws0 = msg_split(layers[0])
    h, ta, tb = _embed_pre(xp, params['emb_W'], _v(params['emb_b']), wd0, ws0)

    for l, p in enumerate(layers):
        edge_w = [
            p['msg_W1'][2 * EMB:2 * EMB + 1], _v(p['msg_b1']),
            _v(p['msg_g1']), _v(p['msg_be1']),
            p['msg_W2'], _v(p['msg_b2']), _v(p['msg_g2']), _v(p['msg_be2']),
            p['pos_W1'], _v(p['pos_b1']), _v(p['pos_g1']), _v(p['pos_be1']),
            p['pos_W2'], _v(p['pos_b2']),
        ]
        node_w = [
            p['upd_W1'][:EMB], p['upd_W1'][EMB:],
            _v(p['upd_b1']), _v(p['upd_g1']), _v(p['upd_be1']),
            p['upd_W2'], _v(p['upd_b2']), _v(p['upd_g2']), _v(p['upd_be2']),
        ]
        gd, gs, pdt = _sc_gather(ta, tb, dst3, src3, pt)
        msg, pmt = _edge(gd, gs, pdt, edge_w)
        agg = _sc_scatter_msg(msg, dst3)
        pagg = _sc_scatter_pos(pmt, dst3)
        if l + 1 < len(layers):
            wd, ws = msg_split(layers[l + 1])
            h, pt, ta, tb = _node(h, pt, agg, pagg, node_w, wd, ws)
        else:
            h, pt, gemb = _node_last(h, pt, agg, pagg, node_w, batch3d)

    return h[:N], gemb, pt[:3, :N].T
